# baseline (speedup 1.0000x reference)
import dataclasses
import functools

import jax
import jax.numpy as jnp
from jax import lax
from jax.experimental import pallas as pl
from jax.experimental.pallas import tpu as pltpu
from jax.experimental.pallas import tpu_sc as plsc

_HIDDEN = 64
_GAMMA = 12.0
_LANES = 16
_NUM_CORES = 2
_NUM_SUBCORES = 16
_NUM_WORKERS = _NUM_CORES * _NUM_SUBCORES
_CHUNK = 128


@functools.partial(jax.jit, static_argnames=("batch",))
def _score(batch, h_idx, r_idx, t_idx, entity_embedding, relation_embedding):
    bpw = batch // _NUM_WORKERS
    n_chunks = bpw // _CHUNK
    mesh = plsc.VectorSubcoreMesh(core_axis_name="c", subcore_axis_name="s")
    cp = pltpu.CompilerParams()
    if "needs_layout_passes" in pltpu.CompilerParams.__dataclass_fields__:
        cp = dataclasses.replace(cp, needs_layout_passes=False)
    if "use_tc_tiling_on_sc" in pltpu.CompilerParams.__dataclass_fields__:
        cp = dataclasses.replace(cp, use_tc_tiling_on_sc=False)

    @functools.partial(
        pl.kernel,
        out_type=jax.ShapeDtypeStruct((batch,), jnp.float32),
        mesh=mesh,
        compiler_params=cp,
        scratch_types=[
            pltpu.VMEM((n_chunks, _CHUNK), jnp.int32),
            pltpu.VMEM((n_chunks, _CHUNK), jnp.int32),
            pltpu.VMEM((n_chunks, _CHUNK), jnp.int32),
            pltpu.VMEM((bpw, _HIDDEN), jnp.float32),
            pltpu.VMEM((bpw, _HIDDEN), jnp.float32),
            pltpu.VMEM((bpw, _HIDDEN), jnp.float32),
            pltpu.VMEM((bpw,), jnp.float32),
            pltpu.SemaphoreType.DMA,
        ],
    )
    def k(ent_hbm, rel_hbm, hi_hbm, ri_hbm, ti_hbm, out_hbm,
          hi_v, ri_v, ti_v, h_v, r_v, t_v, o_v, sem):
        wid = lax.axis_index("s") * _NUM_CORES + lax.axis_index("c")
        row0 = wid * n_chunks

        pltpu.sync_copy(hi_hbm.at[pl.ds(row0, n_chunks)], hi_v)
        pltpu.sync_copy(ri_hbm.at[pl.ds(row0, n_chunks)], ri_v)
        pltpu.sync_copy(ti_hbm.at[pl.ds(row0, n_chunks)], ti_v)

        copies = []
        for c in range(n_chunks):
            dst = pl.ds(c * _CHUNK, _CHUNK)
            copies.append(pltpu.async_copy(ent_hbm.at[hi_v.at[c]], h_v.at[dst], sem))
            copies.append(pltpu.async_copy(rel_hbm.at[ri_v.at[c]], r_v.at[dst], sem))
            copies.append(pltpu.async_copy(ent_hbm.at[ti_v.at[c]], t_v.at[dst], sem))
        for cp in copies:
            cp.wait()

        lane = lax.broadcasted_iota(jnp.int32, (_LANES,), 0)

        @pl.loop(0, bpw, step=_LANES)
        def _(g):
            vec = jnp.zeros((_LANES,), jnp.float32)
            for m in range(_LANES):
                acc = jnp.zeros((_LANES,), jnp.float32)
                for j in range(_HIDDEN // _LANES):
                    sl = pl.ds(j * _LANES, _LANES)
                    acc = acc + jnp.abs(
                        h_v[g + m, sl] + r_v[g + m, sl] - t_v[g + m, sl])
                vec = jnp.where(lane == m, _GAMMA - jnp.sum(acc), vec)
            o_v[pl.ds(g, _LANES)] = vec

        pltpu.sync_copy(o_v, out_hbm.at[pl.ds(wid * bpw, bpw)])

    return k(entity_embedding, relation_embedding, h_idx, r_idx, t_idx)


def kernel(sample, entity_embedding, relation_embedding):
    batch = sample.shape[0]
    rows = batch // _CHUNK
    h_idx = sample[:, 0].reshape(rows, _CHUNK)
    r_idx = sample[:, 1].reshape(rows, _CHUNK)
    t_idx = sample[:, 2].reshape(rows, _CHUNK)
    score = _score(batch, h_idx, r_idx, t_idx, entity_embedding,
                   relation_embedding)
    return score.reshape(batch, 1)
